# Initial kernel scaffold; baseline (speedup 1.0000x reference)
import jax
import jax.numpy as jnp
from jax.experimental import pallas as pl


def kernel(u, train_nids, adj_ent, adj_rel, usr_emb, rel_emb, ent_emb, W, b):
    raise NotImplementedError("write your pallas kernel here")


# speedup vs baseline: 24.9948x; 24.9948x over previous
import functools

import jax
import jax.numpy as jnp
import numpy as np
from jax import lax
from jax.experimental import pallas as pl
from jax.experimental.pallas import tpu as pltpu
from jax.experimental.pallas import tpu_sc as plsc

NUM_ENT = 100000
NUM_REL = 32
DIM = 16
K = 16
B = 16384

NC = 2
NS = 16
NW = NC * NS
BPW = B // NW
H1_PER_W = BPW * K
H1_CHUNK = 1024
H2_CHUNK = 2048

_MESH = dict(core_axis_name="c", subcore_axis_name="s")
_SC_PARAMS = pltpu.CompilerParams(use_tc_tiling_on_sc=False,
                                  needs_layout_passes=False)


def _wid():
    return lax.axis_index("s") * NC + lax.axis_index("c")


def _sc1_body(u_hbm, tn_hbm, usr_hbm, ent_hbm, adje_hbm, adjr_hbm,
              ue_out, v0_out, e1_out, r1_out,
              idx_v, ue_v, v0_v, e1_v, r1_v, sem):
    b0 = _wid() * BPW
    sl = pl.ds(b0, BPW)
    pltpu.sync_copy(u_hbm.at[sl], idx_v)
    pltpu.async_copy(usr_hbm.at[idx_v], ue_v, sem).wait()
    pltpu.sync_copy(ue_v, ue_out.at[sl])
    pltpu.sync_copy(tn_hbm.at[sl], idx_v)
    pltpu.async_copy(ent_hbm.at[idx_v], v0_v, sem).wait()
    pltpu.sync_copy(v0_v, v0_out.at[sl])
    pltpu.async_copy(adje_hbm.at[idx_v], e1_v, sem).wait()
    pltpu.sync_copy(e1_v, e1_out.at[sl])
    pltpu.async_copy(adjr_hbm.at[idx_v], r1_v, sem).wait()
    pltpu.sync_copy(r1_v, r1_out.at[sl])


def _sc1(u, tn, usr_emb, ent_emb, adj_ent, adj_rel):
    f32, i32 = jnp.float32, jnp.int32
    out_type = (
        jax.ShapeDtypeStruct((B, DIM), f32),
        jax.ShapeDtypeStruct((B, DIM), f32),
        jax.ShapeDtypeStruct((B, K), i32),
        jax.ShapeDtypeStruct((B, K), i32),
    )
    scratch = [
        pltpu.VMEM((BPW,), i32),
        pltpu.VMEM((BPW, DIM), f32),
        pltpu.VMEM((BPW, DIM), f32),
        pltpu.VMEM((BPW, K), i32),
        pltpu.VMEM((BPW, K), i32),
        pltpu.SemaphoreType.DMA,
    ]
    run = pl.kernel(_sc1_body, out_type=out_type,
                    mesh=plsc.VectorSubcoreMesh(**_MESH),
                    scratch_types=scratch, compiler_params=_SC_PARAMS)
    return run(u, tn, usr_emb, ent_emb, adj_ent, adj_rel)


def _sc2_body(e1_hbm, r1_hbm, es_hbm, ent_hbm, adje_hbm, adjr_hbm,
              v1_out, e2_out, u1_out, u2_out,
              es_v, r1_v, u1_v, idx_v, v1_v, e2_v, r2_v, u2_v, sem):
    w = _wid()
    b0 = w * BPW
    pltpu.sync_copy(es_hbm.at[pl.ds(b0, BPW)], es_v)
    pltpu.sync_copy(r1_hbm.at[pl.ds(b0, BPW)], r1_v)

    def u1_row(r, _):
        rows = jnp.full((16,), r, jnp.int32)
        u1_v[r] = plsc.load_gather(es_v, [rows, r1_v[r]])
        return 0

    lax.fori_loop(0, BPW, u1_row, 0)
    pltpu.sync_copy(u1_v, u1_out.at[pl.ds(b0, BPW)])

    def chunk(i, _):
        o = w * H1_PER_W + i * H1_CHUNK
        sl = pl.ds(o, H1_CHUNK)
        pltpu.sync_copy(e1_hbm.at[sl], idx_v)
        pltpu.async_copy(ent_hbm.at[idx_v], v1_v, sem).wait()
        pltpu.sync_copy(v1_v, v1_out.at[sl])
        pltpu.async_copy(adje_hbm.at[idx_v], e2_v, sem).wait()
        pltpu.sync_copy(e2_v, e2_out.at[sl])
        pltpu.async_copy(adjr_hbm.at[idx_v], r2_v, sem).wait()

        def u2_row(r, _):
            b_local = (i * H1_CHUNK + r) // K
            rows = jnp.full((16,), b_local, jnp.int32)
            u2_v[r] = plsc.load_gather(es_v, [rows, r2_v[r]])
            return 0

        lax.fori_loop(0, H1_CHUNK, u2_row, 0)
        pltpu.sync_copy(u2_v, u2_out.at[sl])
        return 0

    lax.fori_loop(0, H1_PER_W // H1_CHUNK, chunk, 0)


def _sc2(e1_flat, r1, es, ent_emb, adj_ent, adj_rel):
    f32, i32 = jnp.float32, jnp.int32
    out_type = (
        jax.ShapeDtypeStruct((B * K, DIM), f32),
        jax.ShapeDtypeStruct((B * K, K), i32),
        jax.ShapeDtypeStruct((B, K), f32),
        jax.ShapeDtypeStruct((B * K, K), f32),
    )
    scratch = [
        pltpu.VMEM((BPW, NUM_REL), f32),
        pltpu.VMEM((BPW, K), i32),
        pltpu.VMEM((BPW, K), f32),
        pltpu.VMEM((H1_CHUNK,), i32),
        pltpu.VMEM((H1_CHUNK, DIM), f32),
        pltpu.VMEM((H1_CHUNK, K), i32),
        pltpu.VMEM((H1_CHUNK, K), i32),
        pltpu.VMEM((H1_CHUNK, K), f32),
        pltpu.SemaphoreType.DMA,
    ]
    run = pl.kernel(_sc2_body, out_type=out_type,
                    mesh=plsc.VectorSubcoreMesh(**_MESH),
                    scratch_types=scratch, compiler_params=_SC_PARAMS)
    return run(e1_flat, r1, es, ent_emb, adj_ent, adj_rel)


def _sc3_body(e2_hbm, ent_hbm, v2_out, idx_v, v2_v, sem):
    w = _wid()
    n_per_w = (B * K * K) // NW

    def chunk(i, _):
        sl = pl.ds(w * n_per_w + i * H2_CHUNK, H2_CHUNK)
        pltpu.sync_copy(e2_hbm.at[sl], idx_v)
        pltpu.async_copy(ent_hbm.at[idx_v], v2_v, sem).wait()
        pltpu.sync_copy(v2_v, v2_out.at[sl])
        return 0

    lax.fori_loop(0, n_per_w // H2_CHUNK, chunk, 0)


def _sc3(e2_flat, ent_emb):
    out_type = jax.ShapeDtypeStruct((B * K * K, DIM), jnp.float32)
    scratch = [
        pltpu.VMEM((H2_CHUNK,), jnp.int32),
        pltpu.VMEM((H2_CHUNK, DIM), jnp.float32),
        pltpu.SemaphoreType.DMA,
    ]
    run = pl.kernel(_sc3_body, out_type=out_type,
                    mesh=plsc.VectorSubcoreMesh(**_MESH),
                    scratch_types=scratch, compiler_params=_SC_PARAMS)
    return run(e2_flat, ent_emb)


def _tca_body(ue_ref, relT_ref, es_ref):
    es_ref[...] = jnp.exp(
        jnp.dot(ue_ref[...], relT_ref[...], preferred_element_type=jnp.float32))


def _tca(ue, relT):
    bb = 2048
    return pl.pallas_call(
        _tca_body,
        grid=(B // bb,),
        in_specs=[
            pl.BlockSpec((bb, DIM), lambda i: (i, 0)),
            pl.BlockSpec((DIM, NUM_REL), lambda i: (0, 0)),
        ],
        out_specs=pl.BlockSpec((bb, NUM_REL), lambda i: (i, 0)),
        out_shape=jax.ShapeDtypeStruct((B, NUM_REL), jnp.float32),
    )(ue, relT)


def _sigmoid(x):
    return 1.0 / (1.0 + jnp.exp(-x))


def _tcb1_body(v1_ref, v2_ref, u2_ref, w_ref, b_ref, rmat_ref, g_ref, h1_ref):
    u2 = u2_ref[...]
    usum = jnp.sum(u2, axis=1, keepdims=True)
    n2rep = jnp.dot(u2, rmat_ref[...],
                    preferred_element_type=jnp.float32) / usum
    agg1 = jnp.dot(n2rep * v2_ref[...], g_ref[...],
                   preferred_element_type=jnp.float32)
    x1 = jnp.dot(v1_ref[...] + agg1, w_ref[...],
                 preferred_element_type=jnp.float32) + b_ref[...]
    h1_ref[...] = _sigmoid(x1)


def _tcb1(v1, v2p, u2, W, bvec, rmat, gmat):
    rows = B * K
    rb = 1024
    return pl.pallas_call(
        _tcb1_body,
        grid=(rows // rb,),
        in_specs=[
            pl.BlockSpec((rb, DIM), lambda i: (i, 0)),
            pl.BlockSpec((rb, K * DIM), lambda i: (i, 0)),
            pl.BlockSpec((rb, K), lambda i: (i, 0)),
            pl.BlockSpec((DIM, DIM), lambda i: (0, 0)),
            pl.BlockSpec((1, DIM), lambda i: (0, 0)),
            pl.BlockSpec((K, K * DIM), lambda i: (0, 0)),
            pl.BlockSpec((K * DIM, DIM), lambda i: (0, 0)),
        ],
        out_specs=pl.BlockSpec((rb, DIM), lambda i: (i, 0)),
        out_shape=jax.ShapeDtypeStruct((rows, DIM), jnp.float32),
    )(v1, v2p, u2, W, bvec, rmat, gmat)


def _tcb2_body(ue_ref, v0_ref, u1_ref, v1b_ref, h1b_ref, w_ref, b_ref,
               rmat_ref, g_ref, out_ref):
    u1 = u1_ref[...]
    usum = jnp.sum(u1, axis=1, keepdims=True)
    n1rep = jnp.dot(u1, rmat_ref[...],
                    preferred_element_type=jnp.float32) / usum
    W = w_ref[...]
    bvec = b_ref[...]
    agg0 = jnp.dot(n1rep * v1b_ref[...], g_ref[...],
                   preferred_element_type=jnp.float32)
    h0 = _sigmoid(jnp.dot(v0_ref[...] + agg0, W,
                          preferred_element_type=jnp.float32) + bvec)
    agg0b = jnp.dot(n1rep * h1b_ref[...], g_ref[...],
                    preferred_element_type=jnp.float32)
    t = jnp.dot(h0 + agg0b, W, preferred_element_type=jnp.float32) + bvec
    e2t = jnp.exp(-2.0 * t)
    item = (1.0 - e2t) / (1.0 + e2t)
    sc = jnp.sum(ue_ref[...] * item, axis=1, keepdims=True)
    out_ref[...] = _sigmoid(sc)


def _tcb2(ue, v0, u1, v1b, h1b, W, bvec, rmat, gmat):
    bb = 512
    return pl.pallas_call(
        _tcb2_body,
        grid=(B // bb,),
        in_specs=[
            pl.BlockSpec((bb, DIM), lambda i: (i, 0)),
            pl.BlockSpec((bb, DIM), lambda i: (i, 0)),
            pl.BlockSpec((bb, K), lambda i: (i, 0)),
            pl.BlockSpec((bb, K * DIM), lambda i: (i, 0)),
            pl.BlockSpec((bb, K * DIM), lambda i: (i, 0)),
            pl.BlockSpec((DIM, DIM), lambda i: (0, 0)),
            pl.BlockSpec((1, DIM), lambda i: (0, 0)),
            pl.BlockSpec((K, K * DIM), lambda i: (0, 0)),
            pl.BlockSpec((K * DIM, DIM), lambda i: (0, 0)),
        ],
        out_specs=pl.BlockSpec((bb, 1), lambda i: (i, 0)),
        out_shape=jax.ShapeDtypeStruct((B, 1), jnp.float32),
    )(ue, v0, u1, v1b, h1b, W, bvec, rmat, gmat)


def kernel(u, train_nids, adj_ent, adj_rel, usr_emb, rel_emb, ent_emb, W, b):
    u = u.astype(jnp.int32)
    train_nids = train_nids.astype(jnp.int32)
    adj_ent = adj_ent.astype(jnp.int32)
    adj_rel = adj_rel.astype(jnp.int32)

    ue, v0, e1, r1 = _sc1(u, train_nids, usr_emb, ent_emb, adj_ent, adj_rel)
    es = _tca(ue, rel_emb.T)
    v1, e2, u1, u2 = _sc2(e1.reshape(B * K), r1, es,
                          ent_emb, adj_ent, adj_rel)
    v2 = _sc3(e2.reshape(B * K * K), ent_emb)

    rmat = jnp.asarray(np.kron(np.eye(K, dtype=np.float32),
                               np.ones((1, DIM), np.float32)))
    gmat = jnp.asarray(np.kron(np.ones((K, 1), np.float32),
                               np.eye(DIM, dtype=np.float32)))
    bvec = b.reshape(1, DIM)

    h1 = _tcb1(v1, v2.reshape(B * K, K * DIM), u2, W, bvec, rmat, gmat)
    out = _tcb2(ue, v0, u1, v1.reshape(B, K * DIM), h1.reshape(B, K * DIM),
                W, bvec, rmat, gmat)
    return out.reshape(B)
